# baseline (speedup 1.0000x reference)
import jax
import jax.numpy as jnp
from jax import lax
from jax.experimental import pallas as pl
from jax.experimental.pallas import tpu as pltpu
from jax.experimental.pallas import tpu_sc as plsc

B = 1024
L = 512
D = 64
N = B * L

NC = 2
NS = 16
NW = NC * NS
TPW = N // NW
T = 256
NCHUNK = TPW // T
LD = L * D


def _emb_body(ids2d_hbm, seg_hbm, tok_hbm, pos_hbm, segt_hbm, gb_hbm,
              out_hbm, comb_v, ids_v, segi_v, rows_v, gb_v, sgt_v,
              sem_i, sem_g, sem_o):
    wid = lax.axis_index("s") * NC + lax.axis_index("c")

    def ids_copies(c):
        slot = lax.rem(c, 4)
        gbase = wid * TPW + c * T
        return (
            pltpu.make_async_copy(
                ids2d_hbm.at[pl.ds(wid * (TPW // 128) + c * 2, 2)],
                ids_v.at[slot], sem_i),
            pltpu.make_async_copy(
                seg_hbm.at[pl.ds(gbase, T)],
                segi_v.at[slot, pl.ds(0, T)], sem_i),
        )

    def gather_copies(c):
        b2 = lax.rem(c, 2)
        slot = lax.rem(c, 4)
        return tuple(
            pltpu.make_async_copy(
                tok_hbm.at[ids_v.at[slot, k]],
                rows_v.at[b2, pl.ds(k * 128, 128)], sem_g.at[b2])
            for k in range(2))

    def out_copy(c):
        b2 = lax.rem(c, 2)
        gbase = wid * TPW + c * T
        return pltpu.make_async_copy(
            rows_v.at[b2],
            out_hbm.at[gbase // L, pl.ds(lax.rem(c, 2) * T, T)], sem_o)

    for cp in ids_copies(0):
        cp.start()
    for cp in ids_copies(1):
        cp.start()
    for cp in ids_copies(0):
        cp.wait()
    for cp in gather_copies(0):
        cp.start()

    pltpu.sync_copy(gb_hbm, gb_v)
    pltpu.sync_copy(segt_hbm, sgt_v)
    pltpu.sync_copy(pos_hbm, comb_v.at[pl.ds(0, LD)])
    pltpu.sync_copy(pos_hbm, comb_v.at[pl.ds(LD, LD)])

    s0 = [sgt_v[pl.ds(16 * j, 16)] for j in range(4)]
    s1 = [sgt_v[pl.ds(64 + 16 * j, 16)] for j in range(4)]

    @plsc.parallel_loop(0, L, unroll=4)
    def build(l):
        off = l * D
        for j in range(4):
            o = off + 16 * j
            comb_v[pl.ds(o, 16)] = comb_v[pl.ds(o, 16)] + s0[j]
            o2 = LD + o
            comb_v[pl.ds(o2, 16)] = comb_v[pl.ds(o2, 16)] + s1[j]

    g = [gb_v[pl.ds(16 * j, 16)] for j in range(4)]
    bt = [gb_v[pl.ds(64 + 16 * j, 16)] for j in range(4)]

    def chunk_body(c, carry):
        @pl.when(c >= 1)
        def _():
            out_copy(c - 1).wait()

        @pl.when(c + 1 < NCHUNK)
        def _():
            for cp in ids_copies(c + 1):
                cp.wait()
            for cp in gather_copies(c + 1):
                cp.start()

        @pl.when(c + 2 < NCHUNK)
        def _():
            for cp in ids_copies(c + 2):
                cp.start()

        for cp in gather_copies(c):
            cp.wait()

        b2 = lax.rem(c, 2)
        slot = lax.rem(c, 4)
        lb = b2 * (T * D)

        @plsc.parallel_loop(0, T, unroll=4)
        def token_body(t):
            sid = segi_v[slot, pl.ds(t, 16)][0]
            coff = sid * LD + lb + t * D
            x = [rows_v[b2, t, pl.ds(16 * j, 16)]
                 + comb_v[pl.ds(coff + 16 * j, 16)]
                 for j in range(4)]
            stot = jnp.sum((x[0] + x[1]) + (x[2] + x[3]))
            qtot = jnp.sum((x[0] * x[0] + x[1] * x[1])
                           + (x[2] * x[2] + x[3] * x[3]))
            mean = stot * (1.0 / D)
            var = qtot * (1.0 / D) - mean * mean + 1e-5
            bits = lax.bitcast_convert_type(var, jnp.int32)
            y = lax.bitcast_convert_type(
                jnp.int32(0x5F3759DF) - lax.shift_right_logical(bits, 1),
                jnp.float32)
            for _ in range(3):
                y = y * (1.5 - 0.5 * var * y * y)
            a = y
            nb = mean * a
            for j in range(4):
                rows_v[b2, t, pl.ds(16 * j, 16)] = (x[j] * a - nb) * g[j] + bt[j]

        out_copy(c).start()
        return carry

    lax.fori_loop(0, NCHUNK, chunk_body, 0)
    out_copy(NCHUNK - 1).wait()


@jax.jit
def _emb_call(ids2d, seg_flat, tok_table, pos_flat, segt_flat, gb):
    mesh = plsc.VectorSubcoreMesh(core_axis_name="c", subcore_axis_name="s")
    f = pl.kernel(
        _emb_body,
        out_type=jax.ShapeDtypeStruct((B, L, D), jnp.float32),
        mesh=mesh,
        compiler_params=pltpu.CompilerParams(needs_layout_passes=False,
                                             use_tc_tiling_on_sc=False),
        scratch_types=[
            pltpu.VMEM((2 * LD,), jnp.float32),
            pltpu.VMEM((4, 2, 128), jnp.int32),
            pltpu.VMEM((4, T + 16), jnp.int32),
            pltpu.VMEM((2, T, D), jnp.float32),
            pltpu.VMEM((2 * D,), jnp.float32),
            pltpu.VMEM((2 * D,), jnp.float32),
            pltpu.SemaphoreType.DMA,
            pltpu.SemaphoreType.DMA((2,)),
            pltpu.SemaphoreType.DMA,
        ],
    )
    return f(ids2d, seg_flat, tok_table, pos_flat, segt_flat, gb)


def kernel(input_ids, segment_ids, tok_table, pos_table, seg_table, gamma, beta):
    ids2d = input_ids.astype(jnp.int32).reshape(N // 128, 128)
    seg_flat = segment_ids.astype(jnp.int32).reshape(N)
    pos_flat = pos_table.reshape(LD)
    segt_flat = seg_table.reshape(2 * D)
    gb = jnp.concatenate([gamma, beta]).astype(jnp.float32)
    return _emb_call(ids2d, seg_flat, tok_table, pos_flat, segt_flat, gb)


# speedup vs baseline: 1.3588x; 1.3588x over previous
import jax
import jax.numpy as jnp
from jax import lax
from jax.experimental import pallas as pl
from jax.experimental.pallas import tpu as pltpu
from jax.experimental.pallas import tpu_sc as plsc

B = 1024
L = 512
D = 64
N = B * L

NC = 2
NS = 16
NW = NC * NS
TPW = N // NW
T = 256
NCHUNK = TPW // T
LD = L * D


def _emb_body(ids2d_hbm, seg_hbm, tok_hbm, pos_hbm, segt_hbm, gb_hbm,
              out_hbm, comb_v, ids_v, segi_v, rows_v, gb_v, sgt_v,
              sem_i, sem_g0, sem_g1, sem_o):
    wid = lax.axis_index("s") * NC + lax.axis_index("c")

    def ids_pair(c, slot):
        gbase = wid * TPW + c * T
        return (
            pltpu.make_async_copy(
                ids2d_hbm.at[pl.ds(wid * (TPW // 128) + c * 2, 2)],
                ids_v.at[slot], sem_i),
            pltpu.make_async_copy(
                seg_hbm.at[pl.ds(gbase, T)],
                segi_v.at[slot, pl.ds(0, T)], sem_i),
        )

    def gather_pair(c, slot, b):
        sem = sem_g0 if b == 0 else sem_g1
        return tuple(
            pltpu.make_async_copy(
                tok_hbm.at[ids_v.at[slot, kk]],
                rows_v.at[b, pl.ds(kk * 128, 128)], sem)
            for kk in range(2))

    def out_cp(c, k):
        row = wid * 32 + c // 2
        return pltpu.make_async_copy(
            rows_v.at[k % 2],
            out_hbm.at[row, pl.ds((k % 2) * T, T)], sem_o)

    for cp in ids_pair(0, 0):
        cp.start()
    for cp in ids_pair(1, 1):
        cp.start()
    for cp in ids_pair(0, 0):
        cp.wait()
    for cp in gather_pair(0, 0, 0):
        cp.start()

    pltpu.sync_copy(gb_hbm, gb_v)
    pltpu.sync_copy(segt_hbm, sgt_v)
    pltpu.sync_copy(pos_hbm, comb_v.at[pl.ds(0, LD)])
    pltpu.sync_copy(pos_hbm, comb_v.at[pl.ds(LD, LD)])

    s0 = [sgt_v[pl.ds(16 * j, 16)] for j in range(4)]
    s1 = [sgt_v[pl.ds(64 + 16 * j, 16)] for j in range(4)]

    @plsc.parallel_loop(0, L, unroll=4)
    def build(l):
        off = l * D
        for j in range(4):
            o = off + 16 * j
            comb_v[pl.ds(o, 16)] = comb_v[pl.ds(o, 16)] + s0[j]
            o2 = LD + o
            comb_v[pl.ds(o2, 16)] = comb_v[pl.ds(o2, 16)] + s1[j]

    g = [gb_v[pl.ds(16 * j, 16)] for j in range(4)]
    bt = [gb_v[pl.ds(64 + 16 * j, 16)] for j in range(4)]

    def do_chunk(i, k):
        c = i * 4 + k
        b = k % 2
        lb = b * (T * D)

        if k == 0:
            @pl.when(c >= 1)
            def _():
                out_cp(c - 1, k + 3).wait()
        else:
            out_cp(c - 1, k - 1).wait()

        for cp in gather_pair(c, k, b):
            cp.wait()

        def launch_next():
            for cp in ids_pair(c + 1, (k + 1) % 4):
                cp.wait()
            for cp in gather_pair(c + 1, (k + 1) % 4, (k + 1) % 2):
                cp.start()

        if k == 3:
            @pl.when(c + 1 < NCHUNK)
            def _():
                launch_next()
        else:
            launch_next()

        def prefetch_ids():
            for cp in ids_pair(c + 2, (k + 2) % 4):
                cp.start()

        if k >= 2:
            @pl.when(c + 2 < NCHUNK)
            def _():
                prefetch_ids()
        else:
            prefetch_ids()

        @plsc.parallel_loop(0, T, unroll=4)
        def token_body(t):
            sid = segi_v[k, pl.ds(t, 16)][0]
            coff = sid * LD + lb + t * D
            x = [rows_v[b, t, pl.ds(16 * j, 16)]
                 + comb_v[pl.ds(coff + 16 * j, 16)]
                 for j in range(4)]
            stot = jnp.sum((x[0] + x[1]) + (x[2] + x[3]))
            qtot = jnp.sum((x[0] * x[0] + x[1] * x[1])
                           + (x[2] * x[2] + x[3] * x[3]))
            mean = stot * (1.0 / D)
            var = qtot * (1.0 / D) - mean * mean + 1e-5
            bits = lax.bitcast_convert_type(var, jnp.int32)
            y = lax.bitcast_convert_type(
                jnp.int32(0x5F3759DF) - lax.shift_right_logical(bits, 1),
                jnp.float32)
            for _ in range(3):
                y = y * (1.5 - 0.5 * var * y * y)
            a = y
            nb = mean * a
            for j in range(4):
                rows_v[b, t, pl.ds(16 * j, 16)] = (x[j] * a - nb) * g[j] + bt[j]

        out_cp(c, k).start()

    def body(i, carry):
        for k in range(4):
            do_chunk(i, k)
        return carry

    lax.fori_loop(0, NCHUNK // 4, body, 0)
    out_cp(NCHUNK - 1, 3).wait()


@jax.jit
def _emb_call(ids2d, seg_flat, tok_table, pos_flat, segt_flat, gb):
    mesh = plsc.VectorSubcoreMesh(core_axis_name="c", subcore_axis_name="s")
    f = pl.kernel(
        _emb_body,
        out_type=jax.ShapeDtypeStruct((B, L, D), jnp.float32),
        mesh=mesh,
        compiler_params=pltpu.CompilerParams(needs_layout_passes=False,
                                             use_tc_tiling_on_sc=False),
        scratch_types=[
            pltpu.VMEM((2 * LD,), jnp.float32),
            pltpu.VMEM((4, 2, 128), jnp.int32),
            pltpu.VMEM((4, T + 16), jnp.int32),
            pltpu.VMEM((2, T, D), jnp.float32),
            pltpu.VMEM((2 * D,), jnp.float32),
            pltpu.VMEM((2 * D,), jnp.float32),
            pltpu.SemaphoreType.DMA,
            pltpu.SemaphoreType.DMA,
            pltpu.SemaphoreType.DMA,
            pltpu.SemaphoreType.DMA,
        ],
    )
    return f(ids2d, seg_flat, tok_table, pos_flat, segt_flat, gb)


def kernel(input_ids, segment_ids, tok_table, pos_table, seg_table, gamma, beta):
    ids2d = input_ids.astype(jnp.int32).reshape(N // 128, 128)
    seg_flat = segment_ids.astype(jnp.int32).reshape(N)
    pos_flat = pos_table.reshape(LD)
    segt_flat = seg_table.reshape(2 * D)
    gb = jnp.concatenate([gamma, beta]).astype(jnp.float32)
    return _emb_call(ids2d, seg_flat, tok_table, pos_flat, segt_flat, gb)
